# baseline (speedup 1.0000x reference)
import functools

import jax
import jax.numpy as jnp
from jax import lax
from jax.experimental import pallas as pl
from jax.experimental.pallas import tpu as pltpu
from jax.experimental.pallas import tpu_sc as plsc

B = 16384
F = 26
D = 16
DEEP_IN = F * D
H1 = 400
H2 = 400
N = B * F

NC = 2
NS = 16
NW = NC * NS
PER_W = N // NW
G = 128
CH = 1024
KG = CH // G
NCHUNK = PER_W // CH


def _sc_gather(idx, emb_table, fc_flat):
    mesh = plsc.VectorSubcoreMesh(core_axis_name="c", subcore_axis_name="s")

    @functools.partial(
        pl.kernel,
        out_type=(
            jax.ShapeDtypeStruct((N, D), jnp.float32),
            jax.ShapeDtypeStruct((N,), jnp.float32),
        ),
        mesh=mesh,
        scratch_types=[
            pltpu.VMEM((CH,), jnp.int32),
            pltpu.VMEM((CH, D), jnp.float32),
            pltpu.VMEM((CH,), jnp.float32),
            pltpu.SemaphoreType.DMA,
            pltpu.SemaphoreType.DMA,
        ],
        compiler_params=pltpu.CompilerParams(use_tc_tiling_on_sc=False),
    )
    def body(idx_hbm, emb_hbm, fc_hbm, emb_out, fc_out,
             idx_v, rows_v, fc_v, sem_e, sem_f):
        wid = lax.axis_index("s") * NC + lax.axis_index("c")
        base = wid * PER_W

        def chunk(k, carry):
            off = pl.multiple_of(base + k * CH, CH)
            pltpu.sync_copy(idx_hbm.at[pl.ds(off, CH)], idx_v)
            cps = []
            for j in range(KG):
                sl = pl.ds(j * G, G)
                cps.append(pltpu.async_copy(emb_hbm.at[idx_v.at[sl]],
                                            rows_v.at[sl], sem_e))
                cps.append(pltpu.async_copy(fc_hbm.at[idx_v.at[sl]],
                                            fc_v.at[sl], sem_f))
            for cp in cps:
                cp.wait()
            pltpu.sync_copy(rows_v, emb_out.at[pl.ds(off, CH)])
            pltpu.sync_copy(fc_v, fc_out.at[pl.ds(off, CH)])
            return carry

        lax.fori_loop(0, NCHUNK, chunk, 0)

    return body(idx, emb_table, fc_flat)


BB = 512


def _tc_body(emb_ref, fc_ref, w1_ref, b1_ref, w2_ref, b2_ref, w3_ref,
             wlin_ref, c0_ref, out_ref):
    e = emb_ref[...]
    jj = lax.broadcasted_iota(jnp.int32, (DEEP_IN, D), 0)
    dd = lax.broadcasted_iota(jnp.int32, (DEEP_IN, D), 1)
    s_mat = jnp.where(jj % D == dd, 1.0, 0.0).astype(jnp.float32)
    s = jnp.dot(e, s_mat, preferred_element_type=jnp.float32)
    fm = 0.5 * (jnp.sum(s * s, axis=1, keepdims=True)
                - jnp.sum(e * e, axis=1, keepdims=True))
    lin = (jnp.sum(fc_ref[...], axis=1, keepdims=True) * wlin_ref[0, 0]
           + c0_ref[0, 0])
    h = jnp.dot(e, w1_ref[...], preferred_element_type=jnp.float32,
                precision=lax.Precision.HIGHEST) + b1_ref[...]
    h = jnp.maximum(h, 0.0)
    h = jnp.dot(h, w2_ref[...], preferred_element_type=jnp.float32,
                precision=lax.Precision.HIGHEST) + b2_ref[...]
    h = jnp.maximum(h, 0.0)
    deep = jnp.sum(h * w3_ref[...], axis=1, keepdims=True)
    out_ref[...] = lin + fm + deep


def _tc_compute(emb_flat, fc_b, w1t, b1r, w2t, b2r, w3, wlin, c0):
    return pl.pallas_call(
        _tc_body,
        grid=(B // BB,),
        in_specs=[
            pl.BlockSpec((BB, DEEP_IN), lambda i: (i, 0)),
            pl.BlockSpec((BB, F), lambda i: (i, 0)),
            pl.BlockSpec((DEEP_IN, H1), lambda i: (0, 0)),
            pl.BlockSpec((1, H1), lambda i: (0, 0)),
            pl.BlockSpec((H1, H2), lambda i: (0, 0)),
            pl.BlockSpec((1, H2), lambda i: (0, 0)),
            pl.BlockSpec((1, H2), lambda i: (0, 0)),
            pl.BlockSpec((1, 1), lambda i: (0, 0)),
            pl.BlockSpec((1, 1), lambda i: (0, 0)),
        ],
        out_specs=pl.BlockSpec((BB, 1), lambda i: (i, 0)),
        out_shape=jax.ShapeDtypeStruct((B, 1), jnp.float32),
    )(emb_flat, fc_b, w1t, b1r, w2t, b2r, w3, wlin, c0)


def kernel(x, emb_table, fc_table, W_lin, b_lin, W1, b1, W2, b2, W3, b3):
    idx = x.reshape(N).astype(jnp.int32)
    emb_g, fc_g = _sc_gather(idx, emb_table, fc_table.reshape(-1))
    emb_flat = emb_g.reshape(B, DEEP_IN)
    fc_b = fc_g.reshape(B, F)
    c0 = (b_lin + b3).reshape(1, 1)
    return _tc_compute(emb_flat, fc_b, W1.T, b1.reshape(1, H1), W2.T,
                       b2.reshape(1, H2), W3, W_lin, c0)


# speedup vs baseline: 2.1901x; 2.1901x over previous
import functools

import jax
import jax.numpy as jnp
from jax import lax
from jax.experimental import pallas as pl
from jax.experimental.pallas import tpu as pltpu
from jax.experimental.pallas import tpu_sc as plsc

B = 16384
F = 26
D = 16
DEEP_IN = F * D
H1 = 400
H2 = 400
N = B * F
V = 2600000
VMAIN = 2599936
RM = V // 8

NC = 2
NS = 16
NW = NC * NS
PER_W = N // NW
G = 128
CH = 1024
KG = CH // G
NCHUNK = PER_W // CH

CW = 1024
NCHW = VMAIN // CW


def _sc_convert(emb_t, emb_tail_t):
    mesh = plsc.VectorSubcoreMesh(core_axis_name="c", subcore_axis_name="s")

    @functools.partial(
        pl.kernel,
        out_type=jax.ShapeDtypeStruct((RM, 128), jnp.float32),
        mesh=mesh,
        scratch_types=[
            pltpu.VMEM((16, CW), jnp.float32),
            pltpu.VMEM((128, 128), jnp.float32),
            pltpu.VMEM((16, 64), jnp.float32),
            pltpu.VMEM((8, 128), jnp.float32),
        ],
        compiler_params=pltpu.CompilerParams(needs_layout_passes=False),
    )
    def body(src, tail, out, colbuf, rowbuf, tcol, trow):
        wid = lax.axis_index("s") * NC + lax.axis_index("c")
        nj = jnp.where(wid < NCHW - 79 * NW, 80, 79)
        lanes = lax.iota(jnp.int32, 16)
        rowb = jnp.where(lanes >= 8, 1, 0).astype(jnp.int32)
        colv = [(lanes % 8) * 16 + d for d in range(16)]

        def chunk(j, carry):
            cid = wid + j * NW
            c0 = pl.multiple_of(cid * CW, CW)

            pltpu.sync_copy(src.at[:, pl.ds(c0, CW)], colbuf)

            def grp(g, c2):
                rv = rowb + 2 * g
                for d in range(16):
                    v = colbuf[d, pl.ds(g * 16, 16)]
                    plsc.store_scatter(rowbuf, [rv, colv[d]], v)
                return c2

            lax.fori_loop(0, CW // 16, grp, 0)
            q0 = pl.multiple_of(cid * (CW // 8), CW // 8)
            pltpu.sync_copy(rowbuf, out.at[pl.ds(q0, CW // 8), :])
            return carry

        lax.fori_loop(0, nj, chunk, 0)

        @pl.when(wid == 0)
        def _():
            pltpu.sync_copy(tail, tcol)
            for g in range(4):
                rv = rowb + 2 * g
                for d in range(16):
                    v = tcol[d, pl.ds(g * 16, 16)]
                    plsc.store_scatter(trow, [rv, colv[d]], v)
            pltpu.sync_copy(trow, out.at[pl.ds(VMAIN // 8, 8), :])

    return body(emb_t, emb_tail_t)


def _sc_gather(idx, emb_table, fc_flat):
    mesh = plsc.VectorSubcoreMesh(core_axis_name="c", subcore_axis_name="s")

    @functools.partial(
        pl.kernel,
        out_type=(
            jax.ShapeDtypeStruct((N, D), jnp.float32),
            jax.ShapeDtypeStruct((N,), jnp.float32),
        ),
        mesh=mesh,
        scratch_types=[
            pltpu.VMEM((CH,), jnp.int32),
            pltpu.VMEM((CH, D), jnp.float32),
            pltpu.VMEM((CH,), jnp.float32),
            pltpu.SemaphoreType.DMA,
            pltpu.SemaphoreType.DMA,
        ],
        compiler_params=pltpu.CompilerParams(use_tc_tiling_on_sc=False),
    )
    def body(idx_hbm, emb_hbm, fc_hbm, emb_out, fc_out,
             idx_v, rows_v, fc_v, sem_e, sem_f):
        wid = lax.axis_index("s") * NC + lax.axis_index("c")
        base = wid * PER_W

        def chunk(k, carry):
            off = pl.multiple_of(base + k * CH, CH)
            pltpu.sync_copy(idx_hbm.at[pl.ds(off, CH)], idx_v)
            cps = []
            for j in range(KG):
                sl = pl.ds(j * G, G)
                cps.append(pltpu.async_copy(emb_hbm.at[idx_v.at[sl]],
                                            rows_v.at[sl], sem_e))
                cps.append(pltpu.async_copy(fc_hbm.at[idx_v.at[sl]],
                                            fc_v.at[sl], sem_f))
            for cp in cps:
                cp.wait()
            pltpu.sync_copy(rows_v, emb_out.at[pl.ds(off, CH)])
            pltpu.sync_copy(fc_v, fc_out.at[pl.ds(off, CH)])
            return carry

        lax.fori_loop(0, NCHUNK, chunk, 0)

    return body(idx, emb_table, fc_flat)


BB = 512


def _tc_body(emb_ref, fc_ref, w1_ref, b1_ref, w2_ref, b2_ref, w3_ref,
             wlin_ref, c0_ref, out_ref):
    e = emb_ref[...]
    jj = lax.broadcasted_iota(jnp.int32, (DEEP_IN, D), 0)
    dd = lax.broadcasted_iota(jnp.int32, (DEEP_IN, D), 1)
    s_mat = jnp.where(jj % D == dd, 1.0, 0.0).astype(jnp.float32)
    s = jnp.dot(e, s_mat, preferred_element_type=jnp.float32)
    fm = 0.5 * (jnp.sum(s * s, axis=1, keepdims=True)
                - jnp.sum(e * e, axis=1, keepdims=True))
    lin = (jnp.sum(fc_ref[...], axis=1, keepdims=True) * wlin_ref[0, 0]
           + c0_ref[0, 0])
    h = jnp.dot(e, w1_ref[...], preferred_element_type=jnp.float32) + b1_ref[...]
    h = jnp.maximum(h, 0.0)
    h = jnp.dot(h, w2_ref[...], preferred_element_type=jnp.float32) + b2_ref[...]
    h = jnp.maximum(h, 0.0)
    deep = jnp.sum(h * w3_ref[...], axis=1, keepdims=True)
    out_ref[...] = lin + fm + deep


def _tc_compute(emb_flat, fc_b, w1t, b1r, w2t, b2r, w3, wlin, c0):
    return pl.pallas_call(
        _tc_body,
        grid=(B // BB,),
        in_specs=[
            pl.BlockSpec((BB, DEEP_IN), lambda i: (i, 0)),
            pl.BlockSpec((BB, F), lambda i: (i, 0)),
            pl.BlockSpec((DEEP_IN, H1), lambda i: (0, 0)),
            pl.BlockSpec((1, H1), lambda i: (0, 0)),
            pl.BlockSpec((H1, H2), lambda i: (0, 0)),
            pl.BlockSpec((1, H2), lambda i: (0, 0)),
            pl.BlockSpec((1, H2), lambda i: (0, 0)),
            pl.BlockSpec((1, 1), lambda i: (0, 0)),
            pl.BlockSpec((1, 1), lambda i: (0, 0)),
        ],
        out_specs=pl.BlockSpec((BB, 1), lambda i: (i, 0)),
        out_shape=jax.ShapeDtypeStruct((B, 1), jnp.float32),
    )(emb_flat, fc_b, w1t, b1r, w2t, b2r, w3, wlin, c0)


def kernel(x, emb_table, fc_table, W_lin, b_lin, W1, b1, W2, b2, W3, b3):
    idx = x.reshape(N).astype(jnp.int32)
    table_rm = _sc_convert(emb_table.T, emb_table[VMAIN:, :].T)
    table_lin = table_rm.reshape(V, D)
    emb_g, fc_g = _sc_gather(idx, table_lin, fc_table.reshape(-1))
    emb_flat = emb_g.reshape(B, DEEP_IN)
    fc_b = fc_g.reshape(B, F)
    c0 = (b_lin + b3).reshape(1, 1)
    return _tc_compute(emb_flat, fc_b, W1.T, b1.reshape(1, H1), W2.T,
                       b2.reshape(1, H2), W3, W_lin, c0)


# speedup vs baseline: 2.4735x; 1.1294x over previous
import functools

import jax
import jax.numpy as jnp
from jax import lax
from jax.experimental import pallas as pl
from jax.experimental.pallas import tpu as pltpu
from jax.experimental.pallas import tpu_sc as plsc

B = 16384
F = 26
D = 16
DEEP_IN = F * D
H1 = 400
H2 = 400
N = B * F
V = 2600000
VMAIN = 2599936
RM = V // 8

NC = 2
NS = 16
NW = NC * NS
PER_W = N // NW
G = 128
CH = 1024
KG = CH // G
NCHUNK = PER_W // CH

CW = 1024
NCHW = VMAIN // CW


JW = 80


def _sc_convert(emb_t, emb_tail_t):
    mesh = plsc.VectorSubcoreMesh(core_axis_name="c", subcore_axis_name="s")

    @functools.partial(
        pl.kernel,
        out_type=jax.ShapeDtypeStruct((RM, 128), jnp.float32),
        mesh=mesh,
        scratch_types=[
            pltpu.VMEM((16, CW), jnp.float32),
            pltpu.VMEM((16, CW), jnp.float32),
            pltpu.VMEM((128, 128), jnp.float32),
            pltpu.VMEM((128, 128), jnp.float32),
            pltpu.VMEM((16, 64), jnp.float32),
            pltpu.VMEM((8, 128), jnp.float32),
            pltpu.SemaphoreType.DMA,
            pltpu.SemaphoreType.DMA,
            pltpu.SemaphoreType.DMA,
            pltpu.SemaphoreType.DMA,
        ],
        compiler_params=pltpu.CompilerParams(needs_layout_passes=False),
    )
    def body(src, tail, out, cbuf0, cbuf1, rbuf0, rbuf1, tcol, trow,
             sin0, sin1, sout0, sout1):
        wid = lax.axis_index("s") * NC + lax.axis_index("c")
        lanes = lax.iota(jnp.int32, 16)
        rowb = jnp.where(lanes >= 8, 1, 0).astype(jnp.int32)
        colv = [(lanes % 8) * 16 + d for d in range(16)]

        def c0_of(j):
            return pl.multiple_of((wid * JW + j) * CW, CW)

        def q0_of(j):
            return pl.multiple_of((wid * JW + j) * (CW // 8), CW // 8)

        def valid(j):
            return (wid * JW + j) < NCHW

        def in_cp(j, cbuf, sem):
            return pltpu.make_async_copy(src.at[:, pl.ds(c0_of(j), CW)],
                                         cbuf, sem)

        def out_cp(j, rbuf, sem):
            return pltpu.make_async_copy(rbuf,
                                         out.at[pl.ds(q0_of(j), CW // 8), :],
                                         sem)

        def transpose(cbuf, rbuf):
            for g in range(CW // 16):
                rv = rowb + 2 * g
                for d in range(16):
                    v = cbuf[d, pl.ds(g * 16, 16)]
                    plsc.store_scatter(rbuf, [rv, colv[d]], v)

        @pl.when(valid(0))
        def _():
            in_cp(0, cbuf0, sin0).start()

        @pl.when(valid(1))
        def _():
            in_cp(1, cbuf1, sin1).start()

        def step(jj, carry):
            for half, cbuf, rbuf, si, so in ((0, cbuf0, rbuf0, sin0, sout0),
                                             (1, cbuf1, rbuf1, sin1, sout1)):
                j = 2 * jj + half

                @pl.when(valid(j))
                def _():
                    in_cp(j, cbuf, si).wait()

                @pl.when(valid(j) & (j >= 2))
                def _():
                    out_cp(j - 2, rbuf, so).wait()

                transpose(cbuf, rbuf)

                @pl.when((j + 2 <= JW - 1) & valid(j + 2))
                def _():
                    in_cp(j + 2, cbuf, si).start()

                @pl.when(valid(j))
                def _():
                    out_cp(j, rbuf, so).start()
            return carry

        lax.fori_loop(0, JW // 2, step, 0)

        out_cp(0, rbuf0, sout0).wait()
        out_cp(1, rbuf1, sout1).wait()

        @pl.when(wid == 0)
        def _():
            pltpu.sync_copy(tail, tcol)
            for g in range(4):
                rv = rowb + 2 * g
                for d in range(16):
                    v = tcol[d, pl.ds(g * 16, 16)]
                    plsc.store_scatter(trow, [rv, colv[d]], v)
            pltpu.sync_copy(trow, out.at[pl.ds(VMAIN // 8, 8), :])

    return body(emb_t, emb_tail_t)


def _sc_gather(idx, emb_table, fc_flat):
    mesh = plsc.VectorSubcoreMesh(core_axis_name="c", subcore_axis_name="s")

    @functools.partial(
        pl.kernel,
        out_type=(
            jax.ShapeDtypeStruct((N, D), jnp.float32),
            jax.ShapeDtypeStruct((N,), jnp.float32),
        ),
        mesh=mesh,
        scratch_types=[
            pltpu.VMEM((CH,), jnp.int32),
            pltpu.VMEM((CH, D), jnp.float32),
            pltpu.VMEM((CH,), jnp.float32),
            pltpu.SemaphoreType.DMA,
            pltpu.SemaphoreType.DMA,
        ],
        compiler_params=pltpu.CompilerParams(use_tc_tiling_on_sc=False),
    )
    def body(idx_hbm, emb_hbm, fc_hbm, emb_out, fc_out,
             idx_v, rows_v, fc_v, sem_e, sem_f):
        wid = lax.axis_index("s") * NC + lax.axis_index("c")
        base = wid * PER_W

        def chunk(k, carry):
            off = pl.multiple_of(base + k * CH, CH)
            pltpu.sync_copy(idx_hbm.at[pl.ds(off, CH)], idx_v)
            cps = []
            for j in range(KG):
                sl = pl.ds(j * G, G)
                cps.append(pltpu.async_copy(emb_hbm.at[idx_v.at[sl]],
                                            rows_v.at[sl], sem_e))
                cps.append(pltpu.async_copy(fc_hbm.at[idx_v.at[sl]],
                                            fc_v.at[sl], sem_f))
            for cp in cps:
                cp.wait()
            pltpu.sync_copy(rows_v, emb_out.at[pl.ds(off, CH)])
            pltpu.sync_copy(fc_v, fc_out.at[pl.ds(off, CH)])
            return carry

        lax.fori_loop(0, NCHUNK, chunk, 0)

    return body(idx, emb_table, fc_flat)


BB = 512


def _tc_body(emb_ref, fc_ref, w1_ref, b1_ref, w2_ref, b2_ref, w3_ref,
             wlin_ref, c0_ref, out_ref):
    e = emb_ref[...]
    jj = lax.broadcasted_iota(jnp.int32, (DEEP_IN, D), 0)
    dd = lax.broadcasted_iota(jnp.int32, (DEEP_IN, D), 1)
    s_mat = jnp.where(jj % D == dd, 1.0, 0.0).astype(jnp.float32)
    s = jnp.dot(e, s_mat, preferred_element_type=jnp.float32)
    fm = 0.5 * (jnp.sum(s * s, axis=1, keepdims=True)
                - jnp.sum(e * e, axis=1, keepdims=True))
    lin = (jnp.sum(fc_ref[...], axis=1, keepdims=True) * wlin_ref[0, 0]
           + c0_ref[0, 0])
    h = jnp.dot(e, w1_ref[...], preferred_element_type=jnp.float32) + b1_ref[...]
    h = jnp.maximum(h, 0.0)
    h = jnp.dot(h, w2_ref[...], preferred_element_type=jnp.float32) + b2_ref[...]
    h = jnp.maximum(h, 0.0)
    deep = jnp.sum(h * w3_ref[...], axis=1, keepdims=True)
    out_ref[...] = lin + fm + deep


def _tc_compute(emb_flat, fc_b, w1t, b1r, w2t, b2r, w3, wlin, c0):
    return pl.pallas_call(
        _tc_body,
        grid=(B // BB,),
        in_specs=[
            pl.BlockSpec((BB, DEEP_IN), lambda i: (i, 0)),
            pl.BlockSpec((BB, F), lambda i: (i, 0)),
            pl.BlockSpec((DEEP_IN, H1), lambda i: (0, 0)),
            pl.BlockSpec((1, H1), lambda i: (0, 0)),
            pl.BlockSpec((H1, H2), lambda i: (0, 0)),
            pl.BlockSpec((1, H2), lambda i: (0, 0)),
            pl.BlockSpec((1, H2), lambda i: (0, 0)),
            pl.BlockSpec((1, 1), lambda i: (0, 0)),
            pl.BlockSpec((1, 1), lambda i: (0, 0)),
        ],
        out_specs=pl.BlockSpec((BB, 1), lambda i: (i, 0)),
        out_shape=jax.ShapeDtypeStruct((B, 1), jnp.float32),
    )(emb_flat, fc_b, w1t, b1r, w2t, b2r, w3, wlin, c0)


def kernel(x, emb_table, fc_table, W_lin, b_lin, W1, b1, W2, b2, W3, b3):
    idx = x.reshape(N).astype(jnp.int32)
    table_rm = _sc_convert(emb_table.T, emb_table[VMAIN:, :].T)
    table_lin = table_rm.reshape(V, D)
    emb_g, fc_g = _sc_gather(idx, table_lin, fc_table.reshape(-1))
    emb_flat = emb_g.reshape(B, DEEP_IN)
    fc_b = fc_g.reshape(B, F)
    c0 = (b_lin + b3).reshape(1, 1)
    return _tc_compute(emb_flat, fc_b, W1.T, b1.reshape(1, H1), W2.T,
                       b2.reshape(1, H2), W3, W_lin, c0)
